# baseline (speedup 1.0000x reference)
import functools

import jax
import jax.numpy as jnp
from jax import lax
from jax.experimental import pallas as pl
from jax.experimental.pallas import tpu as pltpu
from jax.experimental.pallas import tpu_sc as plsc

_T = 8192
_SHIFT = 1000
_KEEP = _T - _SHIFT
_ROWS = 16 * 128
_NW = 32
_RPW = _ROWS // _NW


def _sc_roll_body(x_hbm, out_hbm):
    wid = lax.axis_index("s") * 2 + lax.axis_index("c")
    base = wid * _RPW
    pltpu.sync_copy(
        x_hbm.at[pl.ds(base, _RPW), pl.ds(0, _KEEP)],
        out_hbm.at[pl.ds(base, _RPW), pl.ds(_SHIFT, _KEEP)],
    )
    pltpu.sync_copy(
        x_hbm.at[pl.ds(base, _RPW), pl.ds(_KEEP, _SHIFT)],
        out_hbm.at[pl.ds(base, _RPW), pl.ds(0, _SHIFT)],
    )


@jax.jit
def kernel(x):
    rows = x.reshape(_ROWS, _T)
    out = pl.kernel(
        _sc_roll_body,
        out_type=jax.ShapeDtypeStruct((_ROWS, _T), jnp.float32),
        mesh=plsc.VectorSubcoreMesh(core_axis_name="c", subcore_axis_name="s"),
        compiler_params=pltpu.CompilerParams(use_tc_tiling_on_sc=False),
    )(rows)
    return out.reshape(x.shape)


# speedup vs baseline: 168.4927x; 168.4927x over previous
import jax
import jax.numpy as jnp
from jax.experimental import pallas as pl
from jax.experimental.pallas import tpu as pltpu

_T = 8192
_SHIFT = 1000
_ROWS = 16 * 128
_BLOCK_ROWS = 128
_GRID = _ROWS // _BLOCK_ROWS


def _roll_body(x_ref, o_ref):
    o_ref[...] = pltpu.roll(x_ref[...], _SHIFT, axis=1)


@jax.jit
def kernel(x):
    rows = x.reshape(_ROWS, _T)
    out = pl.pallas_call(
        _roll_body,
        grid=(_GRID,),
        in_specs=[pl.BlockSpec((_BLOCK_ROWS, _T), lambda i: (i, 0))],
        out_specs=pl.BlockSpec((_BLOCK_ROWS, _T), lambda i: (i, 0)),
        out_shape=jax.ShapeDtypeStruct((_ROWS, _T), jnp.float32),
    )(rows)
    return out.reshape(x.shape)
